# baseline (speedup 1.0000x reference)
import functools

import jax
import jax.numpy as jnp
from jax import lax
from jax.experimental import pallas as pl
from jax.experimental.pallas import tpu as pltpu
from jax.experimental.pallas import tpu_sc as plsc

_B = 16384
_CB = 49152
_PG = 512
_Q = _PG // 4


def _pack2(a, b):
    return ((a + 0x8000) & jnp.uint32(0xFFFF0000)) | ((b + 0x8000) >> 16)


def _reformat_body(in_ref, out_ref):
    for j in range(_CB // _PG):
        t = jnp.transpose(in_ref[:, j * _PG : (j + 1) * _PG], (1, 0))
        u = lax.bitcast_convert_type(t, jnp.uint32)
        w0 = _pack2(u[:_Q], u[_Q : 2 * _Q])
        w1 = _pack2(u[2 * _Q : 3 * _Q], u[3 * _Q :])
        w = jnp.concatenate([w0, w1], axis=1)
        out_ref[j * _Q : (j + 1) * _Q, :] = lax.bitcast_convert_type(
            w, jnp.float32
        )


def _reformat(tv):
    k, n = tv.shape
    nblk = (n + _CB - 1) // _CB
    return pl.pallas_call(
        _reformat_body,
        grid=(nblk,),
        in_specs=[pl.BlockSpec((k, _CB), lambda i: (0, i))],
        out_specs=pl.BlockSpec((_CB // 4, 128), lambda i: (i, 0)),
        out_shape=jax.ShapeDtypeStruct((nblk * (_CB // 4), 128), jnp.float32),
    )(tv)


def _make_sc_gather(b):
    info = plsc.get_sparse_core_info()
    nc, ns = info.num_cores, info.num_subcores
    nw = nc * ns
    b_per_w = b // nw
    mesh = plsc.VectorSubcoreMesh(core_axis_name="c", subcore_axis_name="s")

    @functools.partial(
        pl.kernel,
        mesh=mesh,
        out_type=jax.ShapeDtypeStruct((b, 128), jnp.float32),
        scratch_types=[
            pltpu.VMEM((b_per_w,), jnp.int32),
            pltpu.VMEM((b_per_w,), jnp.int32),
            pltpu.VMEM((b_per_w, 128), jnp.float32),
            pltpu.SemaphoreType.DMA,
        ],
    )
    def gather_kernel(x_hbm, table_hbm, out_hbm, x_v, row_v, rows_v, sem):
        wid = lax.axis_index("s") * nc + lax.axis_index("c")
        base = wid * b_per_w
        pltpu.sync_copy(x_hbm.at[pl.ds(base, b_per_w)], x_v)

        def cvt(i, carry):
            xc = x_v[pl.ds(i * 16, 16)]
            row_v[pl.ds(i * 16, 16)] = (xc >> 9) * _Q + (xc & (_Q - 1))
            return carry

        lax.fori_loop(0, b_per_w // 16, cvt, 0)
        pltpu.async_copy(table_hbm.at[row_v], rows_v, sem).wait()
        pltpu.sync_copy(rows_v, out_hbm.at[pl.ds(base, b_per_w)])

    return gather_kernel


def _mm_body(pair_ref, half_ref, sub_ref, w_ref, b_ref, out_ref):
    pair = pair_ref[...]
    half = half_ref[...]
    sub = sub_ref[...]
    w64 = jnp.where(half == 0, pair[:, :64], pair[:, 64:])
    u = lax.bitcast_convert_type(w64, jnp.uint32)
    bits = jnp.where(sub == 0, u & jnp.uint32(0xFFFF0000), u << 16)
    emb = lax.bitcast_convert_type(bits, jnp.float32)
    out_ref[...] = (
        jnp.dot(emb, w_ref[...], preferred_element_type=jnp.float32) + b_ref[...]
    )


@jax.jit
def kernel(x, bank, W, b):
    x = jnp.squeeze(x).astype(jnp.int32)
    d = bank.shape[1]
    out_dim = W.shape[1]

    tv = jnp.transpose(bank)
    table = _reformat(tv)

    q = (x % _PG) // _Q
    half = (q >= 2).astype(jnp.int32)
    sub = (q & 1).astype(jnp.int32)

    pair_rows = _make_sc_gather(_B)(x, table)

    blk = 4096
    out = pl.pallas_call(
        _mm_body,
        grid=(_B // blk,),
        in_specs=[
            pl.BlockSpec((blk, 2 * d), lambda i: (i, 0)),
            pl.BlockSpec((blk, 1), lambda i: (i, 0)),
            pl.BlockSpec((blk, 1), lambda i: (i, 0)),
            pl.BlockSpec((d, out_dim), lambda i: (0, 0)),
            pl.BlockSpec((1, out_dim), lambda i: (0, 0)),
        ],
        out_specs=pl.BlockSpec((blk, out_dim), lambda i: (i, 0)),
        out_shape=jax.ShapeDtypeStruct((_B, out_dim), jnp.float32),
    )(pair_rows, half.reshape(_B, 1), sub.reshape(_B, 1), W, b.reshape(1, out_dim))
    return out


# speedup vs baseline: 1.0235x; 1.0235x over previous
import functools

import jax
import jax.numpy as jnp
from jax import lax
from jax.experimental import pallas as pl
from jax.experimental.pallas import tpu as pltpu
from jax.experimental.pallas import tpu_sc as plsc

_B = 16384
_CB = 32768
_PG = 512
_Q = _PG // 4


def _pack2(a, b):
    return ((a + 0x8000) & jnp.uint32(0xFFFF0000)) | ((b + 0x8000) >> 16)


def _reformat_body(in_ref, out_ref):
    for j in range(_CB // _PG):
        t = jnp.transpose(in_ref[:, j * _PG : (j + 1) * _PG], (1, 0))
        u = lax.bitcast_convert_type(t, jnp.uint32)
        w0 = _pack2(u[:_Q], u[_Q : 2 * _Q])
        w1 = _pack2(u[2 * _Q : 3 * _Q], u[3 * _Q :])
        w = jnp.concatenate([w0, w1], axis=1)
        out_ref[j * _Q : (j + 1) * _Q, :] = lax.bitcast_convert_type(
            w, jnp.float32
        )


def _reformat(tv):
    k, n = tv.shape
    nblk = (n + _CB - 1) // _CB
    return pl.pallas_call(
        _reformat_body,
        grid=(nblk,),
        in_specs=[pl.BlockSpec((k, _CB), lambda i: (0, i))],
        out_specs=pl.BlockSpec((_CB // 4, 128), lambda i: (i, 0)),
        out_shape=jax.ShapeDtypeStruct((nblk * (_CB // 4), 128), jnp.float32),
    )(tv)


def _make_sc_gather(b):
    info = plsc.get_sparse_core_info()
    nc, ns = info.num_cores, info.num_subcores
    nw = nc * ns
    b_per_w = b // nw
    mesh = plsc.VectorSubcoreMesh(core_axis_name="c", subcore_axis_name="s")

    @functools.partial(
        pl.kernel,
        mesh=mesh,
        out_type=jax.ShapeDtypeStruct((b, 128), jnp.float32),
        scratch_types=[
            pltpu.VMEM((b_per_w,), jnp.int32),
            pltpu.VMEM((b_per_w,), jnp.int32),
            pltpu.VMEM((b_per_w, 128), jnp.float32),
            pltpu.SemaphoreType.DMA,
        ],
    )
    def gather_kernel(x_hbm, table_hbm, out_hbm, x_v, row_v, rows_v, sem):
        wid = lax.axis_index("s") * nc + lax.axis_index("c")
        base = wid * b_per_w
        pltpu.sync_copy(x_hbm.at[pl.ds(base, b_per_w)], x_v)

        def cvt(i, carry):
            xc = x_v[pl.ds(i * 16, 16)]
            row_v[pl.ds(i * 16, 16)] = (xc >> 9) * _Q + (xc & (_Q - 1))
            return carry

        lax.fori_loop(0, b_per_w // 16, cvt, 0)
        pltpu.async_copy(table_hbm.at[row_v], rows_v, sem).wait()
        pltpu.sync_copy(rows_v, out_hbm.at[pl.ds(base, b_per_w)])

    return gather_kernel


def _mm_body(pair_ref, half_ref, sub_ref, w_ref, b_ref, out_ref):
    pair = pair_ref[...]
    half = half_ref[...]
    sub = sub_ref[...]
    w64 = jnp.where(half == 0, pair[:, :64], pair[:, 64:])
    u = lax.bitcast_convert_type(w64, jnp.uint32)
    bits = jnp.where(sub == 0, u & jnp.uint32(0xFFFF0000), u << 16)
    emb = lax.bitcast_convert_type(bits, jnp.float32)
    out_ref[...] = (
        jnp.dot(emb, w_ref[...], preferred_element_type=jnp.float32) + b_ref[...]
    )


@jax.jit
def kernel(x, bank, W, b):
    x = jnp.squeeze(x).astype(jnp.int32)
    d = bank.shape[1]
    out_dim = W.shape[1]

    tv = jnp.transpose(bank)
    table = _reformat(tv)

    q = (x % _PG) // _Q
    half = (q >= 2).astype(jnp.int32)
    sub = (q & 1).astype(jnp.int32)

    pair_rows = _make_sc_gather(_B)(x, table)

    blk = 4096
    out = pl.pallas_call(
        _mm_body,
        grid=(_B // blk,),
        in_specs=[
            pl.BlockSpec((blk, 2 * d), lambda i: (i, 0)),
            pl.BlockSpec((blk, 1), lambda i: (i, 0)),
            pl.BlockSpec((blk, 1), lambda i: (i, 0)),
            pl.BlockSpec((d, out_dim), lambda i: (0, 0)),
            pl.BlockSpec((1, out_dim), lambda i: (0, 0)),
        ],
        out_specs=pl.BlockSpec((blk, out_dim), lambda i: (i, 0)),
        out_shape=jax.ShapeDtypeStruct((_B, out_dim), jnp.float32),
    )(pair_rows, half.reshape(_B, 1), sub.reshape(_B, 1), W, b.reshape(1, out_dim))
    return out


# speedup vs baseline: 1.0393x; 1.0154x over previous
import functools

import jax
import jax.numpy as jnp
from jax import lax
from jax.experimental import pallas as pl
from jax.experimental.pallas import tpu as pltpu
from jax.experimental.pallas import tpu_sc as plsc

_B = 16384
_CB = 32768
_PG = 512
_Q = _PG // 4


def _pack2(a, b):
    return ((a + 0x8000) & jnp.uint32(0xFFFF0000)) | ((b + 0x8000) >> 16)


def _reformat_body(in_ref, out_ref):
    for j in range(_CB // _PG):
        t = jnp.transpose(in_ref[:, j * _PG : (j + 1) * _PG], (1, 0))
        u = lax.bitcast_convert_type(t, jnp.uint32)
        w0 = _pack2(u[:_Q], u[_Q : 2 * _Q])
        w1 = _pack2(u[2 * _Q : 3 * _Q], u[3 * _Q :])
        w = jnp.concatenate([w0, w1], axis=1)
        out_ref[j * _Q : (j + 1) * _Q, :] = lax.bitcast_convert_type(
            w, jnp.float32
        )


def _reformat(tv):
    k, n = tv.shape
    nblk = (n + _CB - 1) // _CB
    return pl.pallas_call(
        _reformat_body,
        grid=(nblk,),
        in_specs=[pl.BlockSpec((k, _CB), lambda i: (0, i))],
        out_specs=pl.BlockSpec((_CB // 4, 128), lambda i: (i, 0)),
        out_shape=jax.ShapeDtypeStruct((nblk * (_CB // 4), 128), jnp.float32),
    )(tv)


def _make_sc_gather(b):
    info = plsc.get_sparse_core_info()
    nc, ns = info.num_cores, info.num_subcores
    nw = nc * ns
    b_per_w = b // nw
    mesh = plsc.VectorSubcoreMesh(core_axis_name="c", subcore_axis_name="s")

    @functools.partial(
        pl.kernel,
        mesh=mesh,
        out_type=jax.ShapeDtypeStruct((b, 128), jnp.float32),
        scratch_types=[
            pltpu.VMEM((b_per_w,), jnp.int32),
            pltpu.VMEM((b_per_w,), jnp.int32),
            pltpu.VMEM((b_per_w, 128), jnp.float32),
            pltpu.SemaphoreType.DMA,
        ],
    )
    def gather_kernel(x_hbm, table_hbm, out_hbm, x_v, row_v, rows_v, sem):
        wid = lax.axis_index("s") * nc + lax.axis_index("c")
        base = wid * b_per_w
        pltpu.sync_copy(x_hbm.at[pl.ds(base, b_per_w)], x_v)

        def cvt(i, carry):
            xc = x_v[pl.ds(i * 16, 16)]
            row_v[pl.ds(i * 16, 16)] = (xc >> 9) * _Q + (xc & (_Q - 1))
            return carry

        lax.fori_loop(0, b_per_w // 16, cvt, 0)
        pltpu.async_copy(table_hbm.at[row_v], rows_v, sem).wait()
        pltpu.sync_copy(rows_v, out_hbm.at[pl.ds(base, b_per_w)])

    return gather_kernel


def _mm_body(pair_ref, sel_ref, w_ref, b_ref, out_ref):
    pair = pair_ref[...]
    sel = sel_ref[...]
    half = sel >= 2
    sub = sel & 1
    w64 = jnp.where(~half, pair[:, :64], pair[:, 64:])
    u = lax.bitcast_convert_type(w64, jnp.uint32)
    bits = jnp.where(sub == 0, u & jnp.uint32(0xFFFF0000), u << 16)
    emb = lax.bitcast_convert_type(bits, jnp.float32)
    out_ref[...] = (
        jnp.dot(emb, w_ref[...], preferred_element_type=jnp.float32) + b_ref[...]
    )


@jax.jit
def kernel(x, bank, W, b):
    x = jnp.squeeze(x).astype(jnp.int32)
    d = bank.shape[1]
    out_dim = W.shape[1]

    tv = jnp.transpose(bank)
    table = _reformat(tv)

    sel = ((x % _PG) // _Q).astype(jnp.int32)

    pair_rows = _make_sc_gather(_B)(x, table)

    blk = 4096
    out = pl.pallas_call(
        _mm_body,
        grid=(_B // blk,),
        in_specs=[
            pl.BlockSpec((blk, 2 * d), lambda i: (i, 0)),
            pl.BlockSpec((blk, 1), lambda i: (i, 0)),
            pl.BlockSpec((d, out_dim), lambda i: (0, 0)),
            pl.BlockSpec((1, out_dim), lambda i: (0, 0)),
        ],
        out_specs=pl.BlockSpec((blk, out_dim), lambda i: (i, 0)),
        out_shape=jax.ShapeDtypeStruct((_B, out_dim), jnp.float32),
    )(pair_rows, sel.reshape(_B, 1), W, b.reshape(1, out_dim))
    return out


# speedup vs baseline: 1.0397x; 1.0004x over previous
import functools

import jax
import jax.numpy as jnp
from jax import lax
from jax.experimental import pallas as pl
from jax.experimental.pallas import tpu as pltpu
from jax.experimental.pallas import tpu_sc as plsc

_B = 16384
_CB = 32768
_PG = 512
_Q = _PG // 4


def _pack2(a, b):
    return ((a + 0x8000) & jnp.uint32(0xFFFF0000)) | ((b + 0x8000) >> 16)


def _reformat_body(in_ref, out_ref):
    for j in range(_CB // _PG):
        t = jnp.transpose(in_ref[:, j * _PG : (j + 1) * _PG], (1, 0))
        u = lax.bitcast_convert_type(t, jnp.uint32)
        w0 = _pack2(u[:_Q], u[_Q : 2 * _Q])
        w1 = _pack2(u[2 * _Q : 3 * _Q], u[3 * _Q :])
        w = jnp.concatenate([w0, w1], axis=1)
        out_ref[j * _Q : (j + 1) * _Q, :] = lax.bitcast_convert_type(
            w, jnp.float32
        )


def _reformat(tv):
    k, n = tv.shape
    nblk = (n + _CB - 1) // _CB
    return pl.pallas_call(
        _reformat_body,
        grid=(nblk,),
        in_specs=[pl.BlockSpec((k, _CB), lambda i: (0, i))],
        out_specs=pl.BlockSpec((_CB // 4, 128), lambda i: (i, 0)),
        out_shape=jax.ShapeDtypeStruct((nblk * (_CB // 4), 128), jnp.float32),
    )(tv)


def _make_sc_gather(b):
    info = plsc.get_sparse_core_info()
    nc, ns = info.num_cores, info.num_subcores
    nw = nc * ns
    b_per_w = b // nw
    mesh = plsc.VectorSubcoreMesh(core_axis_name="c", subcore_axis_name="s")

    @functools.partial(
        pl.kernel,
        mesh=mesh,
        out_type=jax.ShapeDtypeStruct((b, 128), jnp.float32),
        scratch_types=[
            pltpu.VMEM((b_per_w,), jnp.int32),
            pltpu.VMEM((b_per_w,), jnp.int32),
            pltpu.VMEM((b_per_w, 128), jnp.float32),
            pltpu.SemaphoreType.DMA,
        ],
    )
    def gather_kernel(x_hbm, table_hbm, out_hbm, x_v, row_v, rows_v, sem):
        wid = lax.axis_index("s") * nc + lax.axis_index("c")
        base = wid * b_per_w
        pltpu.sync_copy(x_hbm.at[pl.ds(base, b_per_w)], x_v)

        def cvt(i, carry):
            xc = x_v[pl.ds(i * 16, 16)]
            row_v[pl.ds(i * 16, 16)] = (xc >> 9) * _Q + (xc & (_Q - 1))
            return carry

        lax.fori_loop(0, b_per_w // 16, cvt, 0)
        pltpu.async_copy(table_hbm.at[row_v], rows_v, sem).wait()
        pltpu.sync_copy(rows_v, out_hbm.at[pl.ds(base, b_per_w)])

    return gather_kernel


def _mm_body(pair_ref, sel_ref, w_ref, b_ref, out_ref):
    pair = pair_ref[...]
    sel = sel_ref[...]
    half = sel >= 2
    sub = sel & 1
    w64 = jnp.where(~half, pair[:, :64], pair[:, 64:])
    u = lax.bitcast_convert_type(w64, jnp.uint32)
    bits = jnp.where(sub == 0, u & jnp.uint32(0xFFFF0000), u << 16)
    emb = lax.bitcast_convert_type(bits, jnp.float32)
    out_ref[...] = (
        jnp.dot(emb, w_ref[...], preferred_element_type=jnp.float32) + b_ref[...]
    )


@jax.jit
def kernel(x, bank, W, b):
    x = jnp.squeeze(x).astype(jnp.int32)
    d = bank.shape[1]
    out_dim = W.shape[1]

    tv = jnp.transpose(bank)
    table = _reformat(tv)

    sel = ((x % _PG) // _Q).astype(jnp.int32)

    pair_rows = _make_sc_gather(_B)(x, table)

    blk = 8192
    out = pl.pallas_call(
        _mm_body,
        grid=(_B // blk,),
        in_specs=[
            pl.BlockSpec((blk, 2 * d), lambda i: (i, 0)),
            pl.BlockSpec((blk, 1), lambda i: (i, 0)),
            pl.BlockSpec((d, out_dim), lambda i: (0, 0)),
            pl.BlockSpec((1, out_dim), lambda i: (0, 0)),
        ],
        out_specs=pl.BlockSpec((blk, out_dim), lambda i: (i, 0)),
        out_shape=jax.ShapeDtypeStruct((_B, out_dim), jnp.float32),
    )(pair_rows, sel.reshape(_B, 1), W, b.reshape(1, out_dim))
    return out
